# Initial kernel scaffold; baseline (speedup 1.0000x reference)
import jax
import jax.numpy as jnp
from jax.experimental import pallas as pl


def kernel(x, edge_index, W1, W2, Wd1, bd1, Wd2, bd2, Wo, bo):
    raise NotImplementedError("write your pallas kernel here")


# speedup vs baseline: 22.6342x; 22.6342x over previous
import functools

import jax
import jax.numpy as jnp
from jax import lax
from jax.experimental import pallas as pl
from jax.experimental.pallas import tpu as pltpu
from jax.experimental.pallas import tpu_sc as plsc

N = 10000
E = 320000
D = 128
NP = 10240
NCORE = 2
NSUB = 16
NW = NCORE * NSUB
K = 125
C = E // (NW * K)
RPS = NP // NSUB

_MESH = dict(core_axis_name="c", subcore_axis_name="s")


@functools.partial(
    pl.kernel,
    out_type=jax.ShapeDtypeStruct((NCORE, NP), jnp.float32),
    mesh=plsc.VectorSubcoreMesh(**_MESH),
    scratch_types=[
        pltpu.VMEM((C, K), jnp.int32),
        pltpu.VMEM((128,), jnp.float32),
        pltpu.VMEM((RPS,), jnp.float32),
        pltpu.VMEM_SHARED((NP,), jnp.float32),
    ],
)
def _sc_degree(dst_hbm, out_hbm, dstbuf, ones, zbuf, acc):
    c = lax.axis_index("c")
    s = lax.axis_index("s")
    wid = c * NSUB + s
    for i in range(128 // 16):
        ones[pl.ds(i * 16, 16)] = jnp.full((16,), 1.0, jnp.float32)

    def zfill(i, _):
        zbuf[pl.ds(i * 16, 16)] = jnp.zeros((16,), jnp.float32)
        return 0

    lax.fori_loop(0, RPS // 16, zfill, 0)
    pltpu.sync_copy(zbuf, acc.at[pl.ds(s * RPS, RPS)])
    pltpu.sync_copy(dst_hbm.at[wid], dstbuf)
    plsc.subcore_barrier()

    def body(j, _):
        pltpu.sync_copy(ones.at[pl.ds(0, K)], acc.at[dstbuf.at[j]], add=True)
        return 0

    lax.fori_loop(0, C, body, 0)
    plsc.subcore_barrier()
    pltpu.sync_copy(acc.at[pl.ds(s * RPS, RPS)], out_hbm.at[c, pl.ds(s * RPS, RPS)])


@functools.partial(
    pl.kernel,
    out_type=jax.ShapeDtypeStruct((NCORE, NP, D), jnp.float32),
    mesh=plsc.VectorSubcoreMesh(**_MESH),
    scratch_types=[
        pltpu.VMEM((C, K), jnp.int32),
        pltpu.VMEM((C, K), jnp.int32),
        pltpu.VMEM((K, D), jnp.float32),
        pltpu.VMEM((16, D), jnp.float32),
        pltpu.VMEM_SHARED((NP, D), jnp.float32),
        pltpu.SemaphoreType.DMA,
    ],
)
def _sc_aggregate(hp_hbm, src_hbm, dst_hbm, out_hbm,
                  srcbuf, dstbuf, rows, zblock, acc, sem):
    c = lax.axis_index("c")
    s = lax.axis_index("s")
    wid = c * NSUB + s
    for r in range(16):
        for i in range(D // 16):
            zblock[r, pl.ds(i * 16, 16)] = jnp.zeros((16,), jnp.float32)

    def zfill(t, _):
        pltpu.sync_copy(zblock, acc.at[pl.ds(s * RPS + t * 16, 16)])
        return 0

    lax.fori_loop(0, RPS // 16, zfill, 0)
    pltpu.sync_copy(src_hbm.at[wid], srcbuf)
    pltpu.sync_copy(dst_hbm.at[wid], dstbuf)
    plsc.subcore_barrier()

    def body(j, _):
        pltpu.async_copy(hp_hbm.at[srcbuf.at[j]], rows, sem).wait()
        pltpu.sync_copy(rows, acc.at[dstbuf.at[j]], add=True)
        return 0

    lax.fori_loop(0, C, body, 0)
    plsc.subcore_barrier()
    pltpu.sync_copy(acc.at[pl.ds(s * RPS, RPS)],
                    out_hbm.at[c, pl.ds(s * RPS, RPS)])


_BLK = 512
_NG = NP // _BLK


def _dinv_of(degp_ref):
    deg = degp_ref[0, :] + degp_ref[1, :] + 1.0
    return lax.rsqrt(deg)


def _scale_matmul_body(x_ref, w_ref, degp_ref, o_ref):
    dinv = _dinv_of(degp_ref)
    h = jnp.dot(x_ref[...], w_ref[...], preferred_element_type=jnp.float32)
    o_ref[...] = h * dinv[:, None]


def _tc_scale_matmul(x, w, degp):
    return pl.pallas_call(
        _scale_matmul_body,
        grid=(_NG,),
        in_specs=[
            pl.BlockSpec((_BLK, D), lambda i: (i, 0)),
            pl.BlockSpec((D, D), lambda i: (0, 0)),
            pl.BlockSpec((NCORE, _BLK), lambda i: (0, i)),
        ],
        out_specs=pl.BlockSpec((_BLK, D), lambda i: (i, 0)),
        out_shape=jax.ShapeDtypeStruct((NP, D), jnp.float32),
    )(x, w, degp)


def _mid_body(part_ref, hp_ref, degp_ref, w_ref, o_ref):
    dinv = _dinv_of(degp_ref)
    a = (part_ref[0] + part_ref[1] + hp_ref[...]) * dinv[:, None]
    a = jnp.maximum(a, 0.0)
    h = jnp.dot(a, w_ref[...], preferred_element_type=jnp.float32)
    o_ref[...] = h * dinv[:, None]


def _tc_mid(part, hp, degp, w):
    return pl.pallas_call(
        _mid_body,
        grid=(_NG,),
        in_specs=[
            pl.BlockSpec((NCORE, _BLK, D), lambda i: (0, i, 0)),
            pl.BlockSpec((_BLK, D), lambda i: (i, 0)),
            pl.BlockSpec((NCORE, _BLK), lambda i: (0, i)),
            pl.BlockSpec((D, D), lambda i: (0, 0)),
        ],
        out_specs=pl.BlockSpec((_BLK, D), lambda i: (i, 0)),
        out_shape=jax.ShapeDtypeStruct((NP, D), jnp.float32),
    )(part, hp, degp, w)


def _head_body(part_ref, hp_ref, degp_ref, wd1_ref, bd1_ref, wd2_ref, bd2_ref,
               wo_ref, bo_ref, o_ref, acc_ref):
    i = pl.program_id(0)
    dinv = _dinv_of(degp_ref)
    a = (part_ref[0] + part_ref[1] + hp_ref[...]) * dinv[:, None]
    a = jnp.maximum(a, 0.0)
    part_sum = jnp.sum(a, axis=0, keepdims=True)

    @pl.when(i == 0)
    def _():
        acc_ref[...] = part_sum

    @pl.when(i > 0)
    def _():
        acc_ref[...] = acc_ref[...] + part_sum

    @pl.when(i == _NG - 1)
    def _():
        g = acc_ref[...] * (1.0 / N)
        g = jnp.maximum(
            jnp.dot(g, wd1_ref[...], preferred_element_type=jnp.float32)
            + bd1_ref[...], 0.0)
        g = jnp.maximum(
            jnp.dot(g, wd2_ref[...], preferred_element_type=jnp.float32)
            + bd2_ref[...], 0.0)
        z = jnp.dot(g, wo_ref[...], preferred_element_type=jnp.float32) + bo_ref[...]
        z = z - jnp.max(z, axis=-1, keepdims=True)
        ez = jnp.exp(z)
        o_ref[...] = ez / jnp.sum(ez, axis=-1, keepdims=True)


def _tc_head(part, hp, degp, wd1, bd1, wd2, bd2, wo, bo):
    return pl.pallas_call(
        _head_body,
        grid=(_NG,),
        in_specs=[
            pl.BlockSpec((NCORE, _BLK, D), lambda i: (0, i, 0)),
            pl.BlockSpec((_BLK, D), lambda i: (i, 0)),
            pl.BlockSpec((NCORE, _BLK), lambda i: (0, i)),
            pl.BlockSpec(wd1.shape, lambda i: (0, 0)),
            pl.BlockSpec(bd1.shape, lambda i: (0, 0)),
            pl.BlockSpec(wd2.shape, lambda i: (0, 0)),
            pl.BlockSpec(bd2.shape, lambda i: (0, 0)),
            pl.BlockSpec(wo.shape, lambda i: (0, 0)),
            pl.BlockSpec(bo.shape, lambda i: (0, 0)),
        ],
        out_specs=pl.BlockSpec((1, 10), lambda i: (0, 0)),
        out_shape=jax.ShapeDtypeStruct((1, 10), jnp.float32),
        scratch_shapes=[pltpu.VMEM((1, D), jnp.float32)],
    )(part, hp, degp, wd1, bd1, wd2, bd2, wo, bo)


def kernel(x, edge_index, W1, W2, Wd1, bd1, Wd2, bd2, Wo, bo):
    ei = edge_index.astype(jnp.int32)
    srcs = ei[0].reshape(NW, C, K)
    dsts = ei[1].reshape(NW, C, K)
    x_pad = jnp.pad(x, ((0, NP - N), (0, 0)))

    degp = _sc_degree(dsts)
    hp1 = _tc_scale_matmul(x_pad, W1, degp)
    part1 = _sc_aggregate(hp1, srcs, dsts)
    hp2 = _tc_mid(part1, hp1, degp, W2)
    part2 = _sc_aggregate(hp2, srcs, dsts)
    return _tc_head(part2, hp2, degp, Wd1, bd1.reshape(1, -1),
                    Wd2, bd2.reshape(1, -1), Wo, bo.reshape(1, -1))
